# baseline (speedup 1.0000x reference)
import functools

import jax
import jax.numpy as jnp
from jax import lax
from jax.experimental import pallas as pl
from jax.experimental.pallas import tpu as pltpu
from jax.experimental.pallas import tpu_sc as plsc

_N = 10000
_E = 320000
_D = 128
_NC = 2
_NS = 16
_NW = _NC * _NS
_GL = 128
_G = _E // _GL
_GBASE = _G // _NW
_GREM = _G % _NW
_NPAD = 10240
_RPT = _NPAD // _NS
_GPP = 26
_NPH = _GBASE // _GPP


def _sc_aggregate(col1d, row1d, x):
  mesh = plsc.VectorSubcoreMesh(core_axis_name="c", subcore_axis_name="s")

  @functools.partial(
      pl.kernel,
      out_type=[
          jax.ShapeDtypeStruct((_NC, _NPAD, _D), jnp.float32),
          jax.ShapeDtypeStruct((_NC, _NPAD), jnp.float32),
      ],
      mesh=mesh,
      scratch_types=[
          pltpu.VMEM((2, _GPP * _GL), jnp.int32),
          pltpu.VMEM((2, _GPP * _GL), jnp.int32),
          pltpu.VMEM((_GL, _D), jnp.float32),
          pltpu.VMEM((_GL, _D), jnp.float32),
          pltpu.VMEM((_GL,), jnp.float32),
          pltpu.VMEM_SHARED((_NPAD, _D), jnp.float32),
          pltpu.VMEM_SHARED((_NPAD,), jnp.float32),
          pltpu.SemaphoreType.DMA,
          pltpu.SemaphoreType.DMA,
          pltpu.SemaphoreType.DMA,
          pltpu.SemaphoreType.DMA,
          pltpu.SemaphoreType.DMA,
      ],
  )
  def k(col_hbm, row_hbm, x_hbm, sup_hbm, deg_hbm,
        colb, rowb, rows0, rows1, ones_v, acc, dacc,
        gsem0, gsem1, ssem, dsem, isem):
    c = lax.axis_index("c")
    s = lax.axis_index("s")
    w = c * _NS + s
    tb = s * _RPT
    w78 = w * _GBASE

    def zbody(r, carry):
      for jj in range(_D // 16):
        rows0[r, pl.ds(jj * 16, 16)] = jnp.zeros((16,), jnp.float32)
      return carry

    lax.fori_loop(0, _GL, zbody, 0)
    for r5 in range(_RPT // _GL):
      pltpu.sync_copy(rows0, acc.at[pl.ds(tb + r5 * _GL, _GL)])
      pltpu.sync_copy(rows0.at[0], dacc.at[pl.ds(tb + r5 * _GL, _GL)])
    for j in range(_GL // 16):
      ones_v[pl.ds(j * 16, 16)] = jnp.full((16,), 1.0, jnp.float32)
    plsc.subcore_barrier()

    pltpu.sync_copy(col_hbm.at[pl.ds(w78 * _GL, _GPP * _GL)], colb.at[0])
    pltpu.sync_copy(row_hbm.at[pl.ds(w78 * _GL, _GPP * _GL)], rowb.at[0])
    pltpu.async_copy(x_hbm.at[colb.at[0, pl.ds(0, _GL)]], rows0, gsem0)

    for p in range(_NPH):
      sl = p % 2
      nsl = (p + 1) % 2
      if p + 1 < _NPH:
        nbase = (w78 + (p + 1) * _GPP) * _GL
        pltpu.async_copy(
            col_hbm.at[pl.ds(nbase, _GPP * _GL)], colb.at[nsl], isem)
        pltpu.async_copy(
            row_hbm.at[pl.ds(nbase, _GPP * _GL)], rowb.at[nsl], isem)

      def body(i, carry, sl=sl):
        o0 = 2 * i * _GL
        o1 = o0 + _GL
        pltpu.async_copy(x_hbm.at[colb.at[sl, pl.ds(o1, _GL)]], rows1, gsem1)
        pltpu.make_async_copy(
            x_hbm.at[colb.at[sl, pl.ds(o0, _GL)]], rows0, gsem0).wait()
        pltpu.async_copy(rows0, acc.at[rowb.at[sl, pl.ds(o0, _GL)]], ssem,
                         add=True)
        pltpu.async_copy(ones_v, dacc.at[rowb.at[sl, pl.ds(o0, _GL)]], dsem,
                         add=True)
        pltpu.make_async_copy(
            x_hbm.at[colb.at[sl, pl.ds(o1, _GL)]], rows1, gsem1).wait()
        pltpu.make_async_copy(rows0, acc.at[rowb.at[sl, pl.ds(o0, _GL)]],
                              ssem).wait()

        @pl.when(i < _GPP // 2 - 1)
        def _():
          pltpu.async_copy(x_hbm.at[colb.at[sl, pl.ds(o1 + _GL, _GL)]],
                           rows0, gsem0)

        pltpu.sync_copy(rows1, acc.at[rowb.at[sl, pl.ds(o1, _GL)]], add=True)
        pltpu.async_copy(ones_v, dacc.at[rowb.at[sl, pl.ds(o1, _GL)]], dsem,
                         add=True)
        return carry

      lax.fori_loop(0, _GPP // 2, body, 0)

      def dbody(i, carry, sl=sl):
        pltpu.make_async_copy(ones_v, dacc.at[rowb.at[sl, pl.ds(0, _GL)]],
                              dsem).wait()
        return carry

      lax.fori_loop(0, _GPP, dbody, 0)

      if p + 1 < _NPH:
        nbase = (w78 + (p + 1) * _GPP) * _GL
        pltpu.make_async_copy(
            col_hbm.at[pl.ds(nbase, _GPP * _GL)], colb.at[nsl], isem).wait()
        pltpu.make_async_copy(
            row_hbm.at[pl.ds(nbase, _GPP * _GL)], rowb.at[nsl], isem).wait()
        pltpu.async_copy(x_hbm.at[colb.at[nsl, pl.ds(0, _GL)]], rows0, gsem0)

    @pl.when(w < _GREM)
    def _():
      rbase = (_NW * _GBASE + w) * _GL
      pltpu.sync_copy(col_hbm.at[pl.ds(rbase, _GL)], colb.at[0, pl.ds(0, _GL)])
      pltpu.sync_copy(row_hbm.at[pl.ds(rbase, _GL)], rowb.at[0, pl.ds(0, _GL)])
      pltpu.async_copy(x_hbm.at[colb.at[0, pl.ds(0, _GL)]], rows0,
                       gsem0).wait()
      pltpu.sync_copy(rows0, acc.at[rowb.at[0, pl.ds(0, _GL)]], add=True)
      pltpu.sync_copy(ones_v, dacc.at[rowb.at[0, pl.ds(0, _GL)]], add=True)

    plsc.subcore_barrier()
    pltpu.sync_copy(acc.at[pl.ds(tb, _RPT)], sup_hbm.at[c, pl.ds(tb, _RPT)])
    pltpu.sync_copy(dacc.at[pl.ds(tb, _RPT)], deg_hbm.at[c, pl.ds(tb, _RPT)])

  return k(col1d, row1d, x)


_BLK = 1000
_NBLK = _N // _BLK


def _tc_combine(x, sup, deg3, weight, bias):
  def body(x_ref, s0, s1, d0, d1, w_ref, b_ref, o_ref):
    deg = jnp.maximum(d0[0] + d1[0], 1.0)
    sup_blk = (s0[0] + s1[0]) / deg
    o_ref[...] = (
        jnp.dot(x_ref[...], w_ref[0:_D, :], preferred_element_type=jnp.float32)
        + jnp.dot(sup_blk, w_ref[_D:2 * _D, :],
                  preferred_element_type=jnp.float32)
        + b_ref[...]
    )

  return pl.pallas_call(
      body,
      grid=(_NBLK,),
      in_specs=[
          pl.BlockSpec((_BLK, _D), lambda i: (i, 0)),
          pl.BlockSpec((1, _BLK, _D), lambda i: (0, i, 0)),
          pl.BlockSpec((1, _BLK, _D), lambda i: (1, i, 0)),
          pl.BlockSpec((1, _BLK, 1), lambda i: (0, i, 0)),
          pl.BlockSpec((1, _BLK, 1), lambda i: (1, i, 0)),
          pl.BlockSpec((2 * _D, _D), lambda i: (0, 0)),
          pl.BlockSpec((1, _D), lambda i: (0, 0)),
      ],
      out_specs=pl.BlockSpec((_BLK, _D), lambda i: (i, 0)),
      out_shape=jax.ShapeDtypeStruct((_N, _D), jnp.float32),
  )(x, sup, sup, deg3, deg3, weight, bias)


def kernel(x, edge_index, weight, bias):
  ei = edge_index.astype(jnp.int32)
  sup, deg = _sc_aggregate(ei[1], ei[0], x)
  out = _tc_combine(x, sup, deg[..., None], weight, bias)
  return (out, edge_index)


# speedup vs baseline: 1.0704x; 1.0704x over previous
import functools

import jax
import jax.numpy as jnp
from jax import lax
from jax.experimental import pallas as pl
from jax.experimental.pallas import tpu as pltpu
from jax.experimental.pallas import tpu_sc as plsc

_N = 10000
_E = 320000
_D = 128
_NC = 2
_NS = 16
_NW = _NC * _NS
_GL = 128
_G = _E // _GL
_GBASE = _G // _NW
_GREM = _G % _NW
_NPAD = 10240
_RPT = _NPAD // _NS
_GPP = 26
_NPH = _GBASE // _GPP


def _sc_aggregate(idx2, x):
  mesh = plsc.VectorSubcoreMesh(core_axis_name="c", subcore_axis_name="s")

  @functools.partial(
      pl.kernel,
      out_type=[
          jax.ShapeDtypeStruct((_NC, _NPAD, _D), jnp.float32),
          jax.ShapeDtypeStruct((_NC, _NPAD), jnp.float32),
      ],
      mesh=mesh,
      scratch_types=[
          pltpu.VMEM((2, _GPP, 2, _GL), jnp.int32),
          pltpu.VMEM((_GL, _D), jnp.float32),
          pltpu.VMEM((_GL, _D), jnp.float32),
          pltpu.VMEM((_GL,), jnp.float32),
          pltpu.VMEM_SHARED((_NPAD, _D), jnp.float32),
          pltpu.VMEM_SHARED((_NPAD,), jnp.float32),
          pltpu.SemaphoreType.DMA,
          pltpu.SemaphoreType.DMA,
          pltpu.SemaphoreType.DMA,
          pltpu.SemaphoreType.DMA,
          pltpu.SemaphoreType.DMA,
      ],
  )
  def k(idx_hbm, x_hbm, sup_hbm, deg_hbm,
        idxall, rows0, rows1, ones_v, acc, dacc,
        gsem0, gsem1, ssem, dsem, isem):
    c = lax.axis_index("c")
    s = lax.axis_index("s")
    w = c * _NS + s
    tb = s * _RPT
    w78 = w * _GBASE

    def zbody(r, carry):
      for jj in range(_D // 16):
        rows0[r, pl.ds(jj * 16, 16)] = jnp.zeros((16,), jnp.float32)
      return carry

    lax.fori_loop(0, _GL, zbody, 0)
    for r5 in range(_RPT // _GL):
      pltpu.sync_copy(rows0, acc.at[pl.ds(tb + r5 * _GL, _GL)])
      pltpu.sync_copy(rows0.at[0], dacc.at[pl.ds(tb + r5 * _GL, _GL)])
    for j in range(_GL // 16):
      ones_v[pl.ds(j * 16, 16)] = jnp.full((16,), 1.0, jnp.float32)
    plsc.subcore_barrier()

    pltpu.sync_copy(idx_hbm.at[pl.ds(w78, _GPP)], idxall.at[0])
    pltpu.async_copy(x_hbm.at[idxall.at[0, 0, 0]], rows0, gsem0)

    for p in range(_NPH):
      sl = p % 2
      nsl = (p + 1) % 2
      if p + 1 < _NPH:
        pltpu.async_copy(
            idx_hbm.at[pl.ds(w78 + (p + 1) * _GPP, _GPP)], idxall.at[nsl],
            isem)

      def body(i, carry, sl=sl):
        j0 = 2 * i
        j1 = j0 + 1
        pltpu.async_copy(x_hbm.at[idxall.at[sl, j1, 0]], rows1, gsem1)
        pltpu.make_async_copy(
            x_hbm.at[idxall.at[sl, j0, 0]], rows0, gsem0).wait()
        pltpu.async_copy(rows0, acc.at[idxall.at[sl, j0, 1]], ssem, add=True)
        pltpu.async_copy(ones_v, dacc.at[idxall.at[sl, j0, 1]], dsem,
                         add=True)
        pltpu.make_async_copy(
            x_hbm.at[idxall.at[sl, j1, 0]], rows1, gsem1).wait()
        pltpu.make_async_copy(rows0, acc.at[idxall.at[sl, j0, 1]],
                              ssem).wait()

        @pl.when(i < _GPP // 2 - 1)
        def _():
          pltpu.async_copy(x_hbm.at[idxall.at[sl, j0 + 2, 0]], rows0, gsem0)

        pltpu.sync_copy(rows1, acc.at[idxall.at[sl, j1, 1]], add=True)
        pltpu.async_copy(ones_v, dacc.at[idxall.at[sl, j1, 1]], dsem,
                         add=True)
        return carry

      lax.fori_loop(0, _GPP // 2, body, 0)

      def dbody(i, carry, sl=sl):
        pltpu.make_async_copy(ones_v, dacc.at[idxall.at[sl, 0, 1]],
                              dsem).wait()
        return carry

      lax.fori_loop(0, _GPP, dbody, 0)

      if p + 1 < _NPH:
        pltpu.make_async_copy(
            idx_hbm.at[pl.ds(w78 + (p + 1) * _GPP, _GPP)], idxall.at[nsl],
            isem).wait()
        pltpu.async_copy(x_hbm.at[idxall.at[nsl, 0, 0]], rows0, gsem0)

    @pl.when(w < _GREM)
    def _():
      pltpu.sync_copy(idx_hbm.at[pl.ds(_NW * _GBASE + w, 1)],
                      idxall.at[0, pl.ds(0, 1)])
      pltpu.async_copy(x_hbm.at[idxall.at[0, 0, 0]], rows0, gsem0).wait()
      pltpu.sync_copy(rows0, acc.at[idxall.at[0, 0, 1]], add=True)
      pltpu.sync_copy(ones_v, dacc.at[idxall.at[0, 0, 1]], add=True)

    plsc.subcore_barrier()
    pltpu.sync_copy(acc.at[pl.ds(tb, _RPT)], sup_hbm.at[c, pl.ds(tb, _RPT)])
    pltpu.sync_copy(dacc.at[pl.ds(tb, _RPT)], deg_hbm.at[c, pl.ds(tb, _RPT)])

  return k(idx2, x)


_BLK = 2000
_NBLK = _N // _BLK


def _tc_combine(x, sup, deg3, weight, bias):
  def body(x_ref, s0, s1, d0, d1, w_ref, b_ref, o_ref):
    deg = jnp.maximum(d0[0] + d1[0], 1.0)
    sup_blk = (s0[0] + s1[0]) / deg
    o_ref[...] = (
        jnp.dot(x_ref[...], w_ref[0:_D, :], preferred_element_type=jnp.float32)
        + jnp.dot(sup_blk, w_ref[_D:2 * _D, :],
                  preferred_element_type=jnp.float32)
        + b_ref[...]
    )

  return pl.pallas_call(
      body,
      grid=(_NBLK,),
      in_specs=[
          pl.BlockSpec((_BLK, _D), lambda i: (i, 0)),
          pl.BlockSpec((1, _BLK, _D), lambda i: (0, i, 0)),
          pl.BlockSpec((1, _BLK, _D), lambda i: (1, i, 0)),
          pl.BlockSpec((1, _BLK, 1), lambda i: (0, i, 0)),
          pl.BlockSpec((1, _BLK, 1), lambda i: (1, i, 0)),
          pl.BlockSpec((2 * _D, _D), lambda i: (0, 0)),
          pl.BlockSpec((1, _D), lambda i: (0, 0)),
      ],
      out_specs=pl.BlockSpec((_BLK, _D), lambda i: (i, 0)),
      out_shape=jax.ShapeDtypeStruct((_N, _D), jnp.float32),
  )(x, sup, sup, deg3, deg3, weight, bias)


def kernel(x, edge_index, weight, bias):
  ei = edge_index.astype(jnp.int32)
  idx2 = jnp.stack([ei[1].reshape(_G, _GL), ei[0].reshape(_G, _GL)], axis=1)
  sup, deg = _sc_aggregate(idx2, x)
  out = _tc_combine(x, sup, deg[..., None], weight, bias)
  return (out, edge_index)
